# baseline (speedup 1.0000x reference)
import functools

import jax
import jax.numpy as jnp
from jax import lax
from jax.experimental import pallas as pl
from jax.experimental.pallas import tpu as pltpu
from jax.experimental.pallas import tpu_sc as plsc

ALPHA = 0.4
VOCAB_P1 = 100001
VPAD = 100352
B = 16384
LROW = 200
NC, NS, LANES = 2, 16, 16
NW = NC * NS
ROWS_PER_W = B // NW
CH = 32
NCHUNK = ROWS_PER_W // CH
NPAIR = NCHUNK // 2
UNROLL = 8


def _fuse_body(w_ref, r_ref, rn_ref, o_ref):
    o_ref[...] = (w_ref[...] + 0.4) * (
        (1.0 - ALPHA) * r_ref[...] + ALPHA * rn_ref[...])


def _fuse_table(w2, r2, rn2):
    return pl.pallas_call(
        _fuse_body,
        out_shape=jax.ShapeDtypeStruct((VPAD // 128, 128), jnp.float32),
    )(w2, r2, rn2)


def _sc_body(tbl_hbm, idx_hbm, c_hbm, out_hbm,
             tbl_v, idx_v0, idx_v1, out_v, c_v, sem_t, sem0, sem1):
    wid = lax.axis_index("s") * NC + lax.axis_index("c")
    base_row = wid * ROWS_PER_W
    row_iota = lax.iota(jnp.int32, LANES)

    def start_idx(ch, buf, sem):
        r0 = base_row + ch * CH
        pltpu.async_copy(idx_hbm.at[pl.ds(r0, CH)], buf, sem)

    def wait_idx(ch, buf, sem):
        r0 = base_row + ch * CH
        pltpu.make_async_copy(idx_hbm.at[pl.ds(r0, CH)], buf, sem).wait()

    h_tbl = pltpu.async_copy(tbl_hbm, tbl_v, sem_t)
    start_idx(0, idx_v0, sem0)
    start_idx(1, idx_v1, sem1)
    pltpu.sync_copy(c_hbm, c_v)
    cvec = c_v[...]
    c0 = cvec[0]
    c1 = cvec[1]
    h_tbl.wait()

    def process(ch, buf):
        for g in range(CH // LANES):
            rows = row_iota + g * LANES

            def l_body(lb, accs):
                a0, a1, a2, a3 = accs
                lbase = lb * UNROLL
                for j in range(UNROLL):
                    lv = row_iota + (lbase + j)
                    lv = jnp.where(lv >= LROW, lv - LROW, lv)
                    idxv = plsc.load_gather(buf, [rows, lv])
                    vals = plsc.load_gather(tbl_v, [idxv])
                    if j % 4 == 0:
                        a0 = a0 + vals
                    elif j % 4 == 1:
                        a1 = a1 + vals
                    elif j % 4 == 2:
                        a2 = a2 + vals
                    else:
                        a3 = a3 + vals
                return a0, a1, a2, a3

            z = jnp.zeros((LANES,), jnp.float32)
            a0, a1, a2, a3 = lax.fori_loop(0, LROW // UNROLL, l_body,
                                           (z, z, z, z))
            s = (a0 + a1) + (a2 + a3)
            opos = (ch * CH + g * LANES + row_iota) * 2
            plsc.store_scatter(out_v, [opos], s * c0)
            plsc.store_scatter(out_v, [opos + 1], s * c1)

    def pair_body(p, carry):
        c_a = p * 2

        wait_idx(c_a, idx_v0, sem0)
        process(c_a, idx_v0)

        @pl.when(p + 1 < NPAIR)
        def _():
            start_idx(c_a + 2, idx_v0, sem0)

        wait_idx(c_a + 1, idx_v1, sem1)
        process(c_a + 1, idx_v1)

        @pl.when(p + 1 < NPAIR)
        def _():
            start_idx(c_a + 3, idx_v1, sem1)

        return carry

    lax.fori_loop(0, NPAIR, pair_body, 0)
    pltpu.sync_copy(out_v, out_hbm.at[pl.ds(base_row * 2, ROWS_PER_W * 2)])


_sc_gather_sum = functools.partial(
    pl.kernel,
    out_type=jax.ShapeDtypeStruct((B * 2,), jnp.float32),
    mesh=plsc.VectorSubcoreMesh(core_axis_name="c", subcore_axis_name="s"),
    compiler_params=pltpu.CompilerParams(needs_layout_passes=False,
                                         disable_bounds_checks=True,
                                         use_tc_tiling_on_sc=True),
    scratch_types=[
        pltpu.VMEM((VPAD,), jnp.float32),
        pltpu.VMEM((CH, LROW), jnp.int32),
        pltpu.VMEM((CH, LROW), jnp.int32),
        pltpu.VMEM((ROWS_PER_W * 2,), jnp.float32),
        pltpu.VMEM((16,), jnp.float32),
        pltpu.SemaphoreType.DMA,
        pltpu.SemaphoreType.DMA,
        pltpu.SemaphoreType.DMA,
    ],
)(_sc_body)


def kernel(feat_idx, w_weight, r_weight, r_noise_weight, fc_weight):
    pad = VPAD - VOCAB_P1
    w2 = jnp.pad(w_weight[:, 0], (0, pad)).reshape(VPAD // 128, 128)
    r2 = jnp.pad(r_weight[:, 0], (0, pad)).reshape(VPAD // 128, 128)
    rn2 = jnp.pad(r_noise_weight[:, 0], (0, pad)).reshape(VPAD // 128, 128)
    tbl = _fuse_table(w2, r2, rn2).reshape(VPAD)
    c = fc_weight[:, 0] - fc_weight[:, 1]
    c16 = jnp.pad(c, (0, 14))
    out_flat = _sc_gather_sum(tbl, feat_idx, c16)
    return out_flat.reshape(B, 2)


# speedup vs baseline: 1.7476x; 1.7476x over previous
import functools

import jax
import jax.numpy as jnp
from jax import lax
from jax.experimental import pallas as pl
from jax.experimental.pallas import tpu as pltpu
from jax.experimental.pallas import tpu_sc as plsc

ALPHA = 0.4
VOCAB_P1 = 100001
VPAD = 100352
B = 16384
LROW = 200
NC, NS, LANES = 2, 16, 16
NW = NC * NS
ROWS_PER_W = B // NW
NCB = ROWS_PER_W // 128
L0, L1 = 104, 96
NCHUNK = NCB * 2


def _fuse_body(w_ref, r_ref, rn_ref, o_ref):
    o_ref[...] = (w_ref[...] + 0.4) * (
        (1.0 - ALPHA) * r_ref[...] + ALPHA * rn_ref[...])


def _fuse_table(w2, r2, rn2):
    return pl.pallas_call(
        _fuse_body,
        out_shape=jax.ShapeDtypeStruct((VPAD // 128, 128), jnp.float32),
    )(w2, r2, rn2)


def _sc_body(tbl_hbm, idxt_hbm, c_hbm, out_hbm,
             tbl_v, idx_v0, idx_v1, out_v, c_v, sem_t, sem0, sem1):
    wid = lax.axis_index("s") * NC + lax.axis_index("c")
    col_base = wid * ROWS_PER_W

    def chunk_refs(k, buf):
        cb, h = k // 2, k % 2
        l0 = h * L0
        ln = L1 if h else L0
        src = idxt_hbm.at[pl.ds(l0, ln), pl.ds(col_base + cb * 128, 128)]
        dst = buf.at[pl.ds(0, ln)]
        return src, dst

    def start_idx(k, buf, sem):
        src, dst = chunk_refs(k, buf)
        pltpu.async_copy(src, dst, sem)

    def wait_idx(k, buf, sem):
        src, dst = chunk_refs(k, buf)
        pltpu.make_async_copy(src, dst, sem).wait()

    h_tbl = pltpu.async_copy(tbl_hbm, tbl_v, sem_t)
    start_idx(0, idx_v0, sem0)
    start_idx(1, idx_v1, sem1)
    pltpu.sync_copy(c_hbm, c_v)
    cvec = c_v[...]
    c0 = cvec[0]
    c1 = cvec[1]
    h_tbl.wait()

    bufs = (idx_v0, idx_v1)
    sems = (sem0, sem1)
    zero = jnp.zeros((LANES,), jnp.float32)
    for cb in range(NCB):
        accs = [zero] * 8
        for h in range(2):
            k = cb * 2 + h
            buf, sem = bufs[k % 2], sems[k % 2]
            wait_idx(k, buf, sem)

            def l_body(l, acc, buf=buf):
                new = []
                for g in range(8):
                    idxv = buf[l, pl.ds(g * LANES, LANES)]
                    vals = plsc.load_gather(tbl_v, [idxv])
                    new.append(acc[g] + vals)
                return tuple(new)

            accs = list(lax.fori_loop(0, L1 if h else L0, l_body,
                                      tuple(accs)))
            if k + 2 < NCHUNK:
                start_idx(k + 2, buf, sem)
        row_iota = lax.iota(jnp.int32, LANES)
        for g in range(8):
            s = accs[g]
            opos = cb * 256 + g * LANES + row_iota
            plsc.store_scatter(out_v, [opos], s * c0)
            plsc.store_scatter(out_v, [opos + 128], s * c1)
    pltpu.sync_copy(out_v, out_hbm.at[pl.ds(wid * (ROWS_PER_W * 2),
                                            ROWS_PER_W * 2)])


_sc_gather_sum = functools.partial(
    pl.kernel,
    out_type=jax.ShapeDtypeStruct((B * 2,), jnp.float32),
    mesh=plsc.VectorSubcoreMesh(core_axis_name="c", subcore_axis_name="s"),
    compiler_params=pltpu.CompilerParams(needs_layout_passes=False,
                                         disable_bounds_checks=True),
    scratch_types=[
        pltpu.VMEM((VPAD,), jnp.float32),
        pltpu.VMEM((L0, 128), jnp.int32),
        pltpu.VMEM((L0, 128), jnp.int32),
        pltpu.VMEM((ROWS_PER_W * 2,), jnp.float32),
        pltpu.VMEM((16,), jnp.float32),
        pltpu.SemaphoreType.DMA,
        pltpu.SemaphoreType.DMA,
        pltpu.SemaphoreType.DMA,
    ],
)(_sc_body)


def kernel(feat_idx, w_weight, r_weight, r_noise_weight, fc_weight):
    pad = VPAD - VOCAB_P1
    w2 = jnp.pad(w_weight[:, 0], (0, pad)).reshape(VPAD // 128, 128)
    r2 = jnp.pad(r_weight[:, 0], (0, pad)).reshape(VPAD // 128, 128)
    rn2 = jnp.pad(r_noise_weight[:, 0], (0, pad)).reshape(VPAD // 128, 128)
    tbl = _fuse_table(w2, r2, rn2).reshape(VPAD)
    c = fc_weight[:, 0] - fc_weight[:, 1]
    c16 = jnp.pad(c, (0, 14))
    out_flat = _sc_gather_sum(tbl, feat_idx.T, c16)
    return (out_flat.reshape(B // 128, 2, 128)
            .transpose(0, 2, 1).reshape(B, 2))


# speedup vs baseline: 1.7989x; 1.0294x over previous
import functools

import jax
import jax.numpy as jnp
from jax import lax
from jax.experimental import pallas as pl
from jax.experimental.pallas import tpu as pltpu
from jax.experimental.pallas import tpu_sc as plsc

ALPHA = 0.4
VOCAB_P1 = 100001
VPAD = 100352
B = 16384
LROW = 200
NC, NS, LANES = 2, 16, 16
NW = NC * NS
ROWS_PER_W = B // NW
NCB = ROWS_PER_W // 128
L0, L1 = 104, 96
NCHUNK = NCB * 2


def _fuse_body(w_ref, r_ref, rn_ref, o_ref):
    o_ref[...] = (w_ref[...] + 0.4) * (
        (1.0 - ALPHA) * r_ref[...] + ALPHA * rn_ref[...])


def _fuse_table(w2, r2, rn2):
    return pl.pallas_call(
        _fuse_body,
        out_shape=jax.ShapeDtypeStruct((VPAD // 128, 128), jnp.float32),
    )(w2, r2, rn2)


def _sc_body(tbl_hbm, idxt_hbm, c_hbm, out_hbm,
             tbl_v, idx_v0, idx_v1, out_v, c_v, sem_t, sem0, sem1):
    wid = lax.axis_index("s") * NC + lax.axis_index("c")
    col_base = wid * ROWS_PER_W

    def chunk_refs(k, buf):
        cb, h = k // 2, k % 2
        l0 = h * L0
        ln = L1 if h else L0
        src = idxt_hbm.at[pl.ds(l0, ln), pl.ds(col_base + cb * 128, 128)]
        dst = buf.at[pl.ds(0, ln)]
        return src, dst

    def start_idx(k, buf, sem):
        src, dst = chunk_refs(k, buf)
        pltpu.async_copy(src, dst, sem)

    def wait_idx(k, buf, sem):
        src, dst = chunk_refs(k, buf)
        pltpu.make_async_copy(src, dst, sem).wait()

    h_tbl = pltpu.async_copy(tbl_hbm, tbl_v, sem_t)
    start_idx(0, idx_v0, sem0)
    start_idx(1, idx_v1, sem1)
    pltpu.sync_copy(c_hbm, c_v)
    cvec = c_v[...]
    c0 = cvec[0]
    c1 = cvec[1]
    h_tbl.wait()

    bufs = (idx_v0, idx_v1)
    sems = (sem0, sem1)
    zero = jnp.zeros((LANES,), jnp.float32)
    for cb in range(NCB):
        accs = [zero] * 8
        for h in range(2):
            k = cb * 2 + h
            buf, sem = bufs[k % 2], sems[k % 2]
            wait_idx(k, buf, sem)

            def l_body(l, acc, buf=buf):
                new = []
                for g in range(8):
                    idxv = buf[l, pl.ds(g * LANES, LANES)]
                    vals = plsc.load_gather(tbl_v, [idxv])
                    new.append(acc[g] + vals)
                return tuple(new)

            accs = list(lax.fori_loop(0, L1 if h else L0, l_body,
                                      tuple(accs)))
            if k + 2 < NCHUNK:
                start_idx(k + 2, buf, sem)
        row_iota = lax.iota(jnp.int32, LANES)
        for g in range(8):
            s = accs[g]
            opos = cb * 256 + g * LANES + row_iota
            plsc.store_scatter(out_v, [opos], s * c0)
            plsc.store_scatter(out_v, [opos + 128], s * c1)
    pltpu.sync_copy(out_v, out_hbm.at[pl.ds(wid * (ROWS_PER_W * 2),
                                            ROWS_PER_W * 2)])


_sc_gather_sum = functools.partial(
    pl.kernel,
    out_type=jax.ShapeDtypeStruct((B * 2,), jnp.float32),
    mesh=plsc.VectorSubcoreMesh(core_axis_name="c", subcore_axis_name="s"),
    compiler_params=pltpu.CompilerParams(needs_layout_passes=False,
                                         disable_bounds_checks=True),
    scratch_types=[
        pltpu.VMEM((VPAD,), jnp.float32),
        pltpu.VMEM((L0, 128), jnp.int32),
        pltpu.VMEM((L0, 128), jnp.int32),
        pltpu.VMEM((ROWS_PER_W * 2,), jnp.float32),
        pltpu.VMEM((16,), jnp.float32),
        pltpu.SemaphoreType.DMA,
        pltpu.SemaphoreType.DMA,
        pltpu.SemaphoreType.DMA,
    ],
)(_sc_body)


def kernel(feat_idx, w_weight, r_weight, r_noise_weight, fc_weight):
    pad = VPAD - VOCAB_P1
    w2 = jnp.pad(w_weight, ((0, pad), (0, 0)))[:, 0].reshape(VPAD // 128, 128)
    r2 = jnp.pad(r_weight, ((0, pad), (0, 0)))[:, 0].reshape(VPAD // 128, 128)
    rn2 = jnp.pad(r_noise_weight, ((0, pad), (0, 0)))[:, 0].reshape(
        VPAD // 128, 128)
    tbl = _fuse_table(w2, r2, rn2).reshape(VPAD)
    c = fc_weight[:, 0] - fc_weight[:, 1]
    c16 = jnp.pad(c, (0, 14))
    out_flat = _sc_gather_sum(tbl, feat_idx.T, c16)
    return (out_flat.reshape(B // 128, 2, 128)
            .transpose(0, 2, 1).reshape(B, 2))


# speedup vs baseline: 1.9349x; 1.0756x over previous
import functools

import jax
import jax.numpy as jnp
from jax import lax
from jax.experimental import pallas as pl
from jax.experimental.pallas import tpu as pltpu
from jax.experimental.pallas import tpu_sc as plsc

ALPHA = 0.4
VOCAB_P1 = 100001
VPAD = 100352
B = 16384
LROW = 200
NC, NS, LANES = 2, 16, 16
NW = NC * NS
ROWS_PER_W = B // NW
NCB = ROWS_PER_W // 128
L0, L1 = 104, 96
NCHUNK = NCB * 2


def _fuse_body(w_ref, r_ref, rn_ref, o_ref):
    def half(lo):
        sl = pl.ds(0 if lo else VPAD // 256, VPAD // 256)
        p = (w_ref[sl] + 0.4) * (
            (1.0 - ALPHA) * r_ref[sl] + ALPHA * rn_ref[sl])
        p16 = lax.convert_element_type(
            lax.convert_element_type(p, jnp.bfloat16), jnp.float32)
        return lax.bitcast_convert_type(p16, jnp.int32)

    ilo, ihi = half(True), half(False)
    o_ref[...] = lax.shift_right_logical(ilo, 16) | (ihi & jnp.int32(-65536))


def _fuse_table(w2, r2, rn2):
    return pl.pallas_call(
        _fuse_body,
        out_shape=jax.ShapeDtypeStruct((VPAD // 256, 128), jnp.int32),
    )(w2, r2, rn2)


def _sc_body(tbl_hbm, idxt_hbm, c_hbm, out_hbm,
             tbl_v, idx_v0, idx_v1, out_v, c_v, sem_t, sem0, sem1):
    wid = lax.axis_index("s") * NC + lax.axis_index("c")
    col_base = wid * ROWS_PER_W

    def chunk_refs(k, buf):
        cb, h = k // 2, k % 2
        l0 = h * L0
        ln = L1 if h else L0
        src = idxt_hbm.at[pl.ds(l0, ln), pl.ds(col_base + cb * 128, 128)]
        dst = buf.at[pl.ds(0, ln)]
        return src, dst

    def start_idx(k, buf, sem):
        src, dst = chunk_refs(k, buf)
        pltpu.async_copy(src, dst, sem)

    def wait_idx(k, buf, sem):
        src, dst = chunk_refs(k, buf)
        pltpu.make_async_copy(src, dst, sem).wait()

    h_tbl = pltpu.async_copy(tbl_hbm, tbl_v, sem_t)
    start_idx(0, idx_v0, sem0)
    start_idx(1, idx_v1, sem1)
    pltpu.sync_copy(c_hbm, c_v)
    cvec = c_v[...]
    c0 = cvec[0]
    c1 = cvec[1]
    h_tbl.wait()

    bufs = (idx_v0, idx_v1)
    sems = (sem0, sem1)
    zero = jnp.zeros((LANES,), jnp.float32)
    for cb in range(NCB):
        accs = [zero] * 8
        for h in range(2):
            k = cb * 2 + h
            buf, sem = bufs[k % 2], sems[k % 2]
            wait_idx(k, buf, sem)

            def l_body(l, acc, buf=buf):
                new = []
                for g in range(8):
                    idxv = buf[l, pl.ds(g * LANES, LANES)]
                    hi = idxv >= (VPAD // 2)
                    k = idxv - jnp.where(hi, VPAD // 2, 0)
                    word = plsc.load_gather(tbl_v, [k])
                    bits = jnp.where(hi, word & jnp.int32(-65536),
                                     lax.shift_left(word, 16))
                    vals = plsc.bitcast(bits, jnp.float32)
                    new.append(acc[g] + vals)
                return tuple(new)

            accs = list(lax.fori_loop(0, L1 if h else L0, l_body,
                                      tuple(accs)))
            if k + 2 < NCHUNK:
                start_idx(k + 2, buf, sem)
        row_iota = lax.iota(jnp.int32, LANES)
        for g in range(8):
            s = accs[g]
            opos = cb * 256 + g * LANES + row_iota
            plsc.store_scatter(out_v, [opos], s * c0)
            plsc.store_scatter(out_v, [opos + 128], s * c1)
    pltpu.sync_copy(out_v, out_hbm.at[pl.ds(wid * (ROWS_PER_W * 2),
                                            ROWS_PER_W * 2)])


_sc_gather_sum = functools.partial(
    pl.kernel,
    out_type=jax.ShapeDtypeStruct((B * 2,), jnp.float32),
    mesh=plsc.VectorSubcoreMesh(core_axis_name="c", subcore_axis_name="s"),
    compiler_params=pltpu.CompilerParams(needs_layout_passes=False,
                                         disable_bounds_checks=True),
    scratch_types=[
        pltpu.VMEM((VPAD // 2,), jnp.int32),
        pltpu.VMEM((L0, 128), jnp.int32),
        pltpu.VMEM((L0, 128), jnp.int32),
        pltpu.VMEM((ROWS_PER_W * 2,), jnp.float32),
        pltpu.VMEM((16,), jnp.float32),
        pltpu.SemaphoreType.DMA,
        pltpu.SemaphoreType.DMA,
        pltpu.SemaphoreType.DMA,
    ],
)(_sc_body)


def kernel(feat_idx, w_weight, r_weight, r_noise_weight, fc_weight):
    pad = VPAD - VOCAB_P1
    w2 = jnp.pad(w_weight, ((0, pad), (0, 0)))[:, 0].reshape(VPAD // 128, 128)
    r2 = jnp.pad(r_weight, ((0, pad), (0, 0)))[:, 0].reshape(VPAD // 128, 128)
    rn2 = jnp.pad(r_noise_weight, ((0, pad), (0, 0)))[:, 0].reshape(
        VPAD // 128, 128)
    tbl = _fuse_table(w2, r2, rn2).reshape(VPAD // 2)
    c = fc_weight[:, 0] - fc_weight[:, 1]
    c16 = jnp.pad(c, (0, 14))
    out_flat = _sc_gather_sum(tbl, feat_idx.T, c16)
    return (out_flat.reshape(B // 128, 2, 128)
            .transpose(0, 2, 1).reshape(B, 2))
